# baseline (speedup 1.0000x reference)
import functools

import jax
import jax.numpy as jnp
from jax import lax
from jax.experimental import pallas as pl
from jax.experimental.pallas import tpu as pltpu
from jax.experimental.pallas import tpu_sc as plsc

F32 = jnp.float32
I32 = jnp.int32

NC, NS, NW = 2, 16, 32
C = 128
CH = 64
TW = 128
NB0, NB1, NB2 = 10000, 5000, 2500
NP0, NP1, NP2 = 10240, 5120, 2560
EC = 64
HC = 32
BLK = 512


def _mesh():
    return plsc.VectorSubcoreMesh(core_axis_name="c", subcore_axis_name="s")


_SC_PARAMS = pltpu.CompilerParams(needs_layout_passes=False)
_GDN = lax.GatherDimensionNumbers(offset_dims=(), collapsed_slice_dims=(0,),
                                  start_index_map=(0,))


def _hist_call(idx, ones, zeros, epad, nacc):
    perw = epad // NW
    iters = perw // HC
    rps = nacc // NS

    @functools.partial(
        pl.kernel, mesh=_mesh(),
        out_type=jax.ShapeDtypeStruct((2 * nacc, C), F32),
        scratch_types=[
            pltpu.VMEM((HC,), I32),
            pltpu.VMEM((HC, C), F32),
            pltpu.VMEM_SHARED((nacc, C), F32),
            pltpu.SemaphoreType.DMA,
        ],
        compiler_params=_SC_PARAMS,
    )
    def k(idx_hbm, ones_hbm, zer_hbm, out_hbm, idx_v, ones_v, acc, sem):
        cid = lax.axis_index("c")
        sid = lax.axis_index("s")
        wid = sid * NC + cid
        r0 = sid * rps
        pltpu.sync_copy(ones_hbm, ones_v)
        pltpu.sync_copy(zer_hbm, acc.at[pl.ds(r0, rps)])
        plsc.subcore_barrier()
        base = wid * perw

        def body(i, carry):
            pltpu.sync_copy(idx_hbm.at[pl.ds(base + i * HC, HC)], idx_v)
            pltpu.sync_copy(ones_v, acc.at[idx_v], add=True)
            return carry

        lax.fori_loop(0, iters, body, 0)
        plsc.subcore_barrier()
        pltpu.sync_copy(acc.at[pl.ds(r0, rps)],
                        out_hbm.at[pl.ds(cid * nacc + r0, rps)])

    return k(idx, ones, zeros)


def _recip_call(hist2, nacc):
    def body(h_ref, o_ref):
        s = h_ref[0:nacc, 0:8] + h_ref[nacc:2 * nacc, 0:8]
        o_ref[...] = 1.0 / jnp.maximum(s, 1.0)

    return pl.pallas_call(
        body,
        out_shape=jax.ShapeDtypeStruct((nacc, 8), F32),
    )(hist2)


def _edge_call(tab2, de, src, dst, zeros, npad, epad):
    perw = epad // NS
    iters = perw // EC
    rps = npad // NS

    @functools.partial(
        pl.kernel, mesh=_mesh(),
        out_type=jax.ShapeDtypeStruct((2 * npad, C), F32),
        scratch_types=[
            pltpu.VMEM((EC,), I32),
            pltpu.VMEM((EC,), I32),
            pltpu.VMEM((EC,), I32),
            pltpu.VMEM((EC,), F32),
            pltpu.VMEM((EC, TW), F32),
            pltpu.VMEM((EC, C), F32),
            pltpu.VMEM((EC,), F32),
            pltpu.VMEM((EC,), F32),
            pltpu.VMEM_SHARED((npad, C), F32),
            pltpu.SemaphoreType.DMA,
        ],
        compiler_params=_SC_PARAMS,
    )
    def k(tab_hbm, de_hbm, src_hbm, dst_hbm, zer_hbm, out_hbm,
          srcv, srcv2, dstv, dv, rowsv, msgv, s0v, s1v, acc, sem):
        cid = lax.axis_index("c")
        sid = lax.axis_index("s")
        r0 = sid * rps
        pltpu.sync_copy(zer_hbm, acc.at[pl.ds(r0, rps)])
        z16 = jnp.zeros((16,), F32)
        for g in range(EC):
            for cb in range(CH // 16):
                msgv[g, pl.ds(CH + cb * 16, 16)] = z16
        plsc.subcore_barrier()
        base = sid * perw
        roff = cid * npad

        def body(i, carry):
            off = base + i * EC
            pltpu.sync_copy(src_hbm.at[pl.ds(off, EC)], srcv)
            pltpu.sync_copy(dst_hbm.at[pl.ds(off, EC)], dstv)
            pltpu.sync_copy(de_hbm.at[pl.ds(off, EC)], dv)

            for g in range(EC // 16):
                srcv2[pl.ds(g * 16, 16)] = srcv[pl.ds(g * 16, 16)] + roff
                d16 = dv[pl.ds(g * 16, 16)]
                q1 = 1.0 / (1.0 + jnp.exp(d16))
                s1v[pl.ds(g * 16, 16)] = q1
                s0v[pl.ds(g * 16, 16)] = 1.0 - q1
            pltpu.async_copy(tab_hbm.at[srcv2], rowsv, sem).wait()

            for g in range(EC // 16):
                a0g = s0v[pl.ds(g * 16, 16)]
                a1g = s1v[pl.ds(g * 16, 16)]
                for kk in range(16):
                    bidx = jnp.full((16, 1), kk, I32)
                    a0 = lax.gather(a0g, bidx, _GDN, (1,),
                                    mode=lax.GatherScatterMode.PROMISE_IN_BOUNDS)
                    a1 = lax.gather(a1g, bidx, _GDN, (1,),
                                    mode=lax.GatherScatterMode.PROMISE_IN_BOUNDS)
                    j = g * 16 + kk
                    for cb in range(CH // 16):
                        v0 = rowsv[j, pl.ds(cb * 16, 16)]
                        v1 = rowsv[j, pl.ds(CH + cb * 16, 16)]
                        msgv[j, pl.ds(cb * 16, 16)] = a0 * v0 + a1 * v1
            pltpu.sync_copy(msgv, acc.at[dstv], add=True)
            return carry

        lax.fori_loop(0, iters, body, 0)
        plsc.subcore_barrier()
        pltpu.sync_copy(acc.at[pl.ds(r0, rps)],
                        out_hbm.at[pl.ds(cid * npad + r0, rps)])

    return k(tab2, de, src, dst, zeros)


def _pool_call(x, cl, zeros, npad_in, npad_out):
    perw = npad_in // NW
    iters = perw // HC
    rps = npad_out // NS

    @functools.partial(
        pl.kernel, mesh=_mesh(),
        out_type=jax.ShapeDtypeStruct((2 * npad_out, C), F32),
        scratch_types=[
            pltpu.VMEM((HC,), I32),
            pltpu.VMEM((HC, C), F32),
            pltpu.VMEM_SHARED((npad_out, C), F32),
            pltpu.SemaphoreType.DMA,
        ],
        compiler_params=_SC_PARAMS,
    )
    def k(x_hbm, cl_hbm, zer_hbm, out_hbm, clv, rowsv, acc, sem):
        cid = lax.axis_index("c")
        sid = lax.axis_index("s")
        wid = sid * NC + cid
        r0 = sid * rps
        pltpu.sync_copy(zer_hbm, acc.at[pl.ds(r0, rps)])
        plsc.subcore_barrier()
        base = wid * perw

        def body(i, carry):
            off = base + i * HC
            pltpu.sync_copy(cl_hbm.at[pl.ds(off, HC)], clv)
            pltpu.sync_copy(x_hbm.at[pl.ds(off, HC)], rowsv)
            pltpu.sync_copy(rowsv, acc.at[clv], add=True)
            return carry

        lax.fori_loop(0, iters, body, 0)
        plsc.subcore_barrier()
        pltpu.sync_copy(acc.at[pl.ds(r0, rps)],
                        out_hbm.at[pl.ds(cid * npad_out + r0, rps)])

    return k(x, cl, zeros)


def _unpool_call(table, idx, npad_out, npad_in):
    perw = npad_out // NW
    iters = perw // HC

    @functools.partial(
        pl.kernel, mesh=_mesh(),
        out_type=jax.ShapeDtypeStruct((npad_out, C), F32),
        scratch_types=[
            pltpu.VMEM((HC,), I32),
            pltpu.VMEM((HC, C), F32),
            pltpu.SemaphoreType.DMA,
        ],
        compiler_params=_SC_PARAMS,
    )
    def k(tab_hbm, idx_hbm, out_hbm, idxv, rowsv, sem):
        cid = lax.axis_index("c")
        sid = lax.axis_index("s")
        wid = sid * NC + cid
        base = wid * perw

        def body(i, carry):
            off = base + i * HC
            pltpu.sync_copy(idx_hbm.at[pl.ds(off, HC)], idxv)
            pltpu.async_copy(tab_hbm.at[idxv], rowsv, sem).wait()
            pltpu.sync_copy(rowsv, out_hbm.at[pl.ds(off, HC)])
            return carry

        lax.fori_loop(0, iters, body, 0)

    return k(table, idx)


def _full(shape):
    return pl.BlockSpec(shape, lambda i: (0,) * len(shape))


def _rows(w):
    return pl.BlockSpec((BLK, w), lambda i: (i, 0))


def _tc_direct_call(xa, xb, walla, wallb, wua, wub, w2a, w2b, npad):
    dual = xb is not None

    def body(*refs):
        if dual:
            (xa_r, xb_r, wa_r, wb_r, wua_r, wub_r, w2a_r, w2b_r,
             t_r, l_r, s_r) = refs
        else:
            xa_r, wa_r, wua_r, w2a_r, t_r, l_r, s_r = refs
        x = xa_r[...]
        y = jnp.dot(x, wa_r[...], preferred_element_type=F32)
        l8 = jnp.dot(x, wua_r[...], preferred_element_type=F32)
        sk = jnp.dot(x, w2a_r[...], preferred_element_type=F32)
        if dual:
            x2 = xb_r[...]
            y = y + jnp.dot(x2, wb_r[...], preferred_element_type=F32)
            l8 = l8 + jnp.dot(x2, wub_r[...], preferred_element_type=F32)
            sk = sk + jnp.dot(x2, w2b_r[...], preferred_element_type=F32)
        t_r[0] = y[:, 0:TW]
        t_r[1] = y[:, TW:2 * TW]
        l_r[...] = l8
        s_r[...] = sk

    ins = [xa] + ([xb] if dual else []) + [walla] + ([wallb] if dual else []) \
        + [wua] + ([wub] if dual else []) + [w2a] + ([w2b] if dual else [])
    in_specs = [_rows(C)] + ([_rows(C)] if dual else []) \
        + [_full((C, 2 * TW))] + ([_full((C, 2 * TW))] if dual else []) \
        + [_full((C, 8))] + ([_full((C, 8))] if dual else []) \
        + [_full((C, C))] + ([_full((C, C))] if dual else [])
    return pl.pallas_call(
        body,
        grid=(npad // BLK,),
        in_specs=in_specs,
        out_specs=[pl.BlockSpec((2, BLK, TW), lambda i: (0, i, 0)),
                   _rows(8), _rows(C)],
        out_shape=[
            jax.ShapeDtypeStruct((2, npad, TW), F32),
            jax.ShapeDtypeStruct((npad, 8), F32),
            jax.ShapeDtypeStruct((npad, C), F32),
        ],
    )(*ins)


def _tc_combine_call(pflat, scale8, bias8, skip, wall, wu, w2, relu, matmul,
                     has_w2, pool, npad):
    has_skip = skip is not None
    nb = npad // BLK

    def body(*refs):
        i = 0
        p0_r = refs[i]; i += 1
        p1_r = refs[i]; i += 1
        sc_r = refs[i]; i += 1
        b_r = refs[i]; i += 1
        sk_r = None
        if has_skip:
            sk_r = refs[i]; i += 1
        if pool:
            agg = p0_r[...] + p1_r[...]
        else:
            agg = jnp.concatenate([p0_r[...][:, 0:CH], p1_r[...][:, 0:CH]],
                                  axis=1)
        x = agg * sc_r[...][:, 0:1] + b_r[0:1, :]
        if has_skip:
            x = x + sk_r[...]
        if relu:
            x = jnp.maximum(x, 0.0)
        if matmul:
            w_r = refs[i]; i += 1
            wu_r = refs[i]; i += 1
            w2_r = None
            if has_w2:
                w2_r = refs[i]; i += 1
            xo_r = refs[i]; i += 1
            t_r = refs[i]; i += 1
            l_r = refs[i]; i += 1
            xo_r[...] = x
            y = jnp.dot(x, w_r[...], preferred_element_type=F32)
            t_r[0] = y[:, 0:TW]
            t_r[1] = y[:, TW:2 * TW]
            l_r[...] = jnp.dot(x, wu_r[...], preferred_element_type=F32)
            if has_w2:
                s_r = refs[i]
                s_r[...] = jnp.dot(x, w2_r[...], preferred_element_type=F32)
        else:
            xo_r = refs[i]
            xo_r[...] = x

    ins = [pflat, pflat, scale8, bias8]
    in_specs = [
        pl.BlockSpec((BLK, C), lambda i: (i, 0)),
        pl.BlockSpec((BLK, C), lambda i: (i + nb, 0)),
        pl.BlockSpec((BLK, 8), lambda i: (i, 0)),
        _full((8, C)),
    ]
    if has_skip:
        ins.append(skip)
        in_specs.append(_rows(C))
    out_specs = [_rows(C)]
    out_shape = [jax.ShapeDtypeStruct((npad, C), F32)]
    if matmul:
        ins += [wall, wu] + ([w2] if has_w2 else [])
        in_specs += [_full((C, 2 * TW)), _full((C, 8))] \
            + ([_full((C, C))] if has_w2 else [])
        out_specs += [pl.BlockSpec((2, BLK, TW), lambda i: (0, i, 0)),
                      _rows(8)] + ([_rows(C)] if has_w2 else [])
        out_shape += [jax.ShapeDtypeStruct((2, npad, TW), F32),
                      jax.ShapeDtypeStruct((npad, 8), F32)] \
            + ([jax.ShapeDtypeStruct((npad, C), F32)] if has_w2 else [])
    return pl.pallas_call(
        body,
        grid=(nb,),
        in_specs=in_specs,
        out_specs=out_specs,
        out_shape=out_shape,
    )(*ins)


def _pad2(a, r, c):
    return jnp.pad(a, ((0, r - a.shape[0]), (0, c - a.shape[1])))


def _pack_wall(p, lo, hi):
    w = p["W"][lo:hi]
    u = p["u"][lo:hi]
    cout = p["b"].shape[0]
    y0 = _pad2(w[:, :cout], C, C)
    y1 = _pad2(w[:, cout:], C, C)
    ud = jnp.pad(u[:, 0] - u[:, 1], (0, C - u.shape[0]))
    wu = jnp.zeros((C, 8), F32).at[:, 0].set(ud)
    blkA = jnp.concatenate([y0[:, 0:CH], y1[:, 0:CH]], axis=1)
    blkB = jnp.concatenate([y0[:, CH:C], y1[:, CH:C]], axis=1)
    wall = jnp.concatenate([blkA, blkB], axis=1)
    w2 = _pad2(p["Ws"][lo:hi], C, C) if "Ws" in p else None
    return wall, wu, w2


def _bias8(p):
    return jnp.broadcast_to(jnp.pad(p["b"], (0, C - p["b"].shape[0])), (8, C))


def _pad_idx(idx, tot, lo, hi):
    n = idx.shape[0]
    fill = lo + (jnp.arange(tot - n, dtype=I32) % (hi - lo))
    return jnp.concatenate([idx.astype(I32), fill])


def kernel(feat, geo, params, scale0_edge_index, edge_index1, edge_index2,
           cluster1, cluster2):
    rows = jnp.array([0, 0, 0, 1, 1, 2])
    cols = jnp.array([0, 1, 2, 1, 2, 2])
    t0 = feat[:, 0][:, rows, cols]
    t1 = feat[:, 1][:, rows, cols]
    t2 = feat[:, 2].reshape(-1, 9)
    x0 = jnp.concatenate([t0, t1, t2, geo[:, None]], axis=1)
    x0 = jnp.pad(x0, ((0, NP0 - NB0), (0, C - x0.shape[1])))

    egrp = NS * EC
    ep0 = egrp * -(-scale0_edge_index.shape[1] // egrp)
    ep1 = egrp * -(-edge_index1.shape[1] // egrp)
    ep2 = egrp * -(-edge_index2.shape[1] // egrp)
    src0 = _pad_idx(scale0_edge_index[0], ep0, 0, NB0)
    dst0 = _pad_idx(scale0_edge_index[1], ep0, NB0, NP0)
    src1 = _pad_idx(edge_index1[0], ep1, 0, NB1)
    dst1 = _pad_idx(edge_index1[1], ep1, NB1, NP1)
    src2 = _pad_idx(edge_index2[0], ep2, 0, NB2)
    dst2 = _pad_idx(edge_index2[1], ep2, NB2, NP2)
    c1p = _pad_idx(cluster1, NP0, NB1, NP1)
    c2p = _pad_idx(cluster2, NP1, NB2, NP2)
    u1p = _pad_idx(cluster2, NP1, 0, NB2)
    u0p = _pad_idx(cluster1, NP0, 0, NB1)

    ones_h = jnp.ones((HC, C), F32)
    z8 = {n: jnp.zeros((n // NS, C), F32) for n in (NP0, NP1, NP2)}
    zF = {n: jnp.zeros((n // NS, C), F32) for n in (NP0, NP1, NP2)}

    hp0 = NW * HC * -(-ep0 // (NW * HC))
    hp1 = NW * HC * -(-ep1 // (NW * HC))
    hp2 = NW * HC * -(-ep2 // (NW * HC))
    dst0h = _pad_idx(dst0, hp0, NB0, NP0)
    dst1h = _pad_idx(dst1, hp1, NB1, NP1)
    dst2h = _pad_idx(dst2, hp2, NB2, NP2)
    invd0 = _recip_call(_hist_call(dst0h, ones_h, z8[NP0], hp0, NP0), NP0)
    invd1 = _recip_call(_hist_call(dst1h, ones_h, z8[NP1], hp1, NP1), NP1)
    invd2 = _recip_call(_hist_call(dst2h, ones_h, z8[NP2], hp2, NP2), NP2)
    invc1 = _recip_call(_hist_call(c1p, ones_h, z8[NP1], NP0, NP1), NP1)
    invc2 = _recip_call(_hist_call(c2p, ones_h, z8[NP2], NP1, NP2), NP2)

    scale = {0: (src0, dst0, ep0, NP0, invd0),
             1: (src1, dst1, ep1, NP1, invd1),
             2: (src2, dst2, ep2, NP2, invd2)}

    def edge(name, tab, l8, s):
        src, dst, ep, npad, _ = scale[s]
        p = params[name]
        cdiff = p["c"][0] - p["c"][1]
        l = l8[:, 0]
        de = l[src] - l[dst] + cdiff
        return _edge_call(tab.reshape(2 * npad, TW), de, src, dst,
                          zF[npad], npad, ep)

    def comb(pf, name, s, skip, relu, matmul, has_w2=False, wname=None):
        _, _, _, npad, invd = scale[s]
        b8 = _bias8(params[name])
        if matmul:
            wall, wu, w2 = _pack_wall(params[wname], 0, C)
            if not has_w2:
                w2 = None
            return _tc_combine_call(pf, invd, b8, skip, wall, wu, w2, relu,
                                    True, has_w2, False, npad)
        return _tc_combine_call(pf, invd, b8, skip, None, None, None, relu,
                                False, False, False, npad)

    def pool_comb(pf, invc, wname, npad):
        wall, wu, _ = _pack_wall(params[wname], 0, C)
        return _tc_combine_call(pf, invc, zb, None, wall, wu, None, False,
                                True, False, True, npad)

    zb = jnp.zeros((8, C), F32)

    wall, wu, w2 = _pack_wall(params["conv01"], 0, C)
    tab, l8, sky = _tc_direct_call(x0, None, wall, None, wu, None, w2, None,
                                   NP0)
    pf = edge("conv01", tab, l8, 0)
    x1, tab, l8 = comb(pf, "conv01", 0, sky, True, True, wname="conv02")
    pf = edge("conv02", tab, l8, 0)
    copy0, = comb(pf, "conv02", 0, x1, True, False)

    pfp = _pool_call(copy0, c1p, zF[NP1], NP0, NP1)
    xp1, tab, l8 = pool_comb(pfp, invc1, "conv11", NP1)
    pf = edge("conv11", tab, l8, 1)
    x11, tab, l8 = comb(pf, "conv11", 1, xp1, True, True, wname="conv12")
    pf = edge("conv12", tab, l8, 1)
    copy1, = comb(pf, "conv12", 1, x11, True, False)

    pfp = _pool_call(copy1, c2p, zF[NP2], NP1, NP2)
    xp2, tab, l8 = pool_comb(pfp, invc2, "conv21", NP2)
    pf = edge("conv21", tab, l8, 2)
    x21, tab, l8 = comb(pf, "conv21", 2, xp2, True, True, wname="conv22")
    pf = edge("conv22", tab, l8, 2)
    x2f, = comb(pf, "conv22", 2, x21, True, False)

    xu1 = _unpool_call(x2f, u1p, NP1, NP2)
    wa, wua, w2a = _pack_wall(params["conv13"], 0, 115)
    wb, wub, w2b = _pack_wall(params["conv13"], 115, 230)
    tab, l8, sky = _tc_direct_call(xu1, copy1, wa, wb, wua, wub, w2a, w2b,
                                   NP1)
    pf = edge("conv13", tab, l8, 1)
    x13, tab, l8 = comb(pf, "conv13", 1, sky, True, True, wname="conv14")
    pf = edge("conv14", tab, l8, 1)
    x14, tab, l8 = comb(pf, "conv14", 1, x13, True, True, wname="conv15")
    pf = edge("conv15", tab, l8, 1)
    x15, tab, l8 = comb(pf, "conv15", 1, x14, True, True, wname="conv16")
    pf = edge("conv16", tab, l8, 1)
    x1f, = comb(pf, "conv16", 1, x15, True, False)

    xu0 = _unpool_call(x1f, u0p, NP0, NP1)
    wa, wua, w2a = _pack_wall(params["conv03"], 0, 115)
    wb, wub, w2b = _pack_wall(params["conv03"], 115, 230)
    tab, l8, sky = _tc_direct_call(xu0, copy0, wa, wb, wua, wub, w2a, w2b,
                                   NP0)
    pf = edge("conv03", tab, l8, 0)
    x03, tab, l8 = comb(pf, "conv03", 0, sky, True, True, wname="conv04")
    pf = edge("conv04", tab, l8, 0)
    x04, tab, l8 = comb(pf, "conv04", 0, x03, True, True, wname="conv05")
    pf = edge("conv05", tab, l8, 0)
    x05, tab, l8, sky = comb(pf, "conv05", 0, x04, True, True, has_w2=True,
                             wname="conv06")
    pf = edge("conv06", tab, l8, 0)
    out, = comb(pf, "conv06", 0, sky, False, False)
    return out[:NB0, :3]


# speedup vs baseline: 1.0497x; 1.0497x over previous
import functools

import jax
import jax.numpy as jnp
from jax import lax
from jax.experimental import pallas as pl
from jax.experimental.pallas import tpu as pltpu
from jax.experimental.pallas import tpu_sc as plsc

F32 = jnp.float32
I32 = jnp.int32

NC, NS, NW = 2, 16, 32
C = 128
CH = 64
TW = 128
NB0, NB1, NB2 = 10000, 5000, 2500
NP0, NP1, NP2 = 10240, 5120, 2560
EC = 64
HC = 32
BLK = 512


def _mesh():
    return plsc.VectorSubcoreMesh(core_axis_name="c", subcore_axis_name="s")


_SC_PARAMS = pltpu.CompilerParams(needs_layout_passes=False)
_GDN = lax.GatherDimensionNumbers(offset_dims=(), collapsed_slice_dims=(0,),
                                  start_index_map=(0,))


def _hist_call(idx, ones, zeros, epad, nacc):
    perw = epad // NW
    iters = perw // HC
    rps = nacc // NS

    @functools.partial(
        pl.kernel, mesh=_mesh(),
        out_type=jax.ShapeDtypeStruct((2 * nacc, C), F32),
        scratch_types=[
            pltpu.VMEM((HC,), I32),
            pltpu.VMEM((HC, C), F32),
            pltpu.VMEM_SHARED((nacc, C), F32),
            pltpu.SemaphoreType.DMA,
        ],
        compiler_params=_SC_PARAMS,
    )
    def k(idx_hbm, ones_hbm, zer_hbm, out_hbm, idx_v, ones_v, acc, sem):
        cid = lax.axis_index("c")
        sid = lax.axis_index("s")
        wid = sid * NC + cid
        r0 = sid * rps
        pltpu.sync_copy(ones_hbm, ones_v)
        pltpu.sync_copy(zer_hbm, acc.at[pl.ds(r0, rps)])
        plsc.subcore_barrier()
        base = wid * perw

        def body(i, carry):
            pltpu.sync_copy(idx_hbm.at[pl.ds(base + i * HC, HC)], idx_v)
            pltpu.sync_copy(ones_v, acc.at[idx_v], add=True)
            return carry

        lax.fori_loop(0, iters, body, 0)
        plsc.subcore_barrier()
        pltpu.sync_copy(acc.at[pl.ds(r0, rps)],
                        out_hbm.at[pl.ds(cid * nacc + r0, rps)])

    return k(idx, ones, zeros)


def _recip_call(hist2, nacc):
    def body(h_ref, o_ref):
        s = h_ref[0:nacc, 0:8] + h_ref[nacc:2 * nacc, 0:8]
        o_ref[...] = 1.0 / jnp.maximum(s, 1.0)

    return pl.pallas_call(
        body,
        out_shape=jax.ShapeDtypeStruct((nacc, 8), F32),
    )(hist2)


def _edge_call(tab2, de, src, dst, zeros, npad, epad):
    perw = epad // NS
    iters = perw // EC
    rps = npad // NS

    @functools.partial(
        pl.kernel, mesh=_mesh(),
        out_type=jax.ShapeDtypeStruct((2 * npad, C), F32),
        scratch_types=[
            pltpu.VMEM((EC,), I32), pltpu.VMEM((EC,), I32),
            pltpu.VMEM((EC,), I32), pltpu.VMEM((EC,), I32),
            pltpu.VMEM((EC,), F32), pltpu.VMEM((EC,), F32),
            pltpu.VMEM((EC, TW), F32), pltpu.VMEM((EC, TW), F32),
            pltpu.VMEM((EC, C), F32),
            pltpu.VMEM((EC,), F32),
            pltpu.VMEM((EC,), F32),
            pltpu.VMEM_SHARED((npad, C), F32),
            pltpu.SemaphoreType.DMA, pltpu.SemaphoreType.DMA,
            pltpu.SemaphoreType.DMA, pltpu.SemaphoreType.DMA,
        ],
        compiler_params=_SC_PARAMS,
    )
    def k(tab_hbm, de_hbm, src_hbm, dst_hbm, zer_hbm, out_hbm,
          srcA, srcB, dstA, dstB, dvA, dvB, rowsA, rowsB, msgv, s0v, s1v,
          acc, semiA, semiB, semgA, semgB, ):
        cid = lax.axis_index("c")
        sid = lax.axis_index("s")
        r0 = sid * rps
        pltpu.sync_copy(zer_hbm, acc.at[pl.ds(r0, rps)])
        z16 = jnp.zeros((16,), F32)
        for g in range(EC):
            for cb in range(CH // 16):
                msgv[g, pl.ds(CH + cb * 16, 16)] = z16
        plsc.subcore_barrier()
        base = sid * perw
        roff = cid * npad
        last = iters - 1

        def fire_idx(sv, dv_, qv, semi, c):
            off = base + jnp.minimum(c, last) * EC
            pltpu.async_copy(src_hbm.at[pl.ds(off, EC)], sv, semi)
            pltpu.async_copy(dst_hbm.at[pl.ds(off, EC)], dv_, semi)
            pltpu.async_copy(de_hbm.at[pl.ds(off, EC)], qv, semi)

        def drain_idx(sv, dv_, qv, semi):
            pltpu.make_async_copy(src_hbm.at[pl.ds(base, EC)], sv, semi).wait()
            pltpu.make_async_copy(dst_hbm.at[pl.ds(base, EC)], dv_, semi).wait()
            pltpu.make_async_copy(de_hbm.at[pl.ds(base, EC)], qv, semi).wait()

        def prep_gather(sv, rows, semg):
            for g in range(EC // 16):
                sv[pl.ds(g * 16, 16)] = sv[pl.ds(g * 16, 16)] + roff
            pltpu.async_copy(tab_hbm.at[sv], rows, semg)

        def drain_gather(sv, rows, semg):
            pltpu.make_async_copy(tab_hbm.at[sv], rows, semg).wait()

        def process(dv_, qv, rows):
            for g in range(EC // 16):
                d16 = qv[pl.ds(g * 16, 16)]
                q1 = 1.0 / (1.0 + jnp.exp(d16))
                s1v[pl.ds(g * 16, 16)] = q1
                s0v[pl.ds(g * 16, 16)] = 1.0 - q1
            for g in range(EC // 16):
                a0g = s0v[pl.ds(g * 16, 16)]
                a1g = s1v[pl.ds(g * 16, 16)]
                for kk in range(16):
                    bidx = jnp.full((16, 1), kk, I32)
                    a0 = lax.gather(a0g, bidx, _GDN, (1,),
                                    mode=lax.GatherScatterMode.PROMISE_IN_BOUNDS)
                    a1 = lax.gather(a1g, bidx, _GDN, (1,),
                                    mode=lax.GatherScatterMode.PROMISE_IN_BOUNDS)
                    j = g * 16 + kk
                    for cb in range(CH // 16):
                        v0 = rows[j, pl.ds(cb * 16, 16)]
                        v1 = rows[j, pl.ds(CH + cb * 16, 16)]
                        msgv[j, pl.ds(cb * 16, 16)] = a0 * v0 + a1 * v1
            pltpu.sync_copy(msgv, acc.at[dv_], add=True)

        fire_idx(srcA, dstA, dvA, semiA, 0)
        drain_idx(srcA, dstA, dvA, semiA)
        prep_gather(srcA, rowsA, semgA)
        fire_idx(srcB, dstB, dvB, semiB, 1)

        def body(i2, carry):
            c = i2 * 2
            drain_idx(srcB, dstB, dvB, semiB)
            prep_gather(srcB, rowsB, semgB)
            drain_gather(srcA, rowsA, semgA)
            process(dstA, dvA, rowsA)
            fire_idx(srcA, dstA, dvA, semiA, c + 2)
            drain_idx(srcA, dstA, dvA, semiA)
            prep_gather(srcA, rowsA, semgA)
            drain_gather(srcB, rowsB, semgB)
            process(dstB, dvB, rowsB)
            fire_idx(srcB, dstB, dvB, semiB, c + 3)
            return carry

        lax.fori_loop(0, iters // 2, body, 0)
        drain_gather(srcA, rowsA, semgA)
        drain_idx(srcB, dstB, dvB, semiB)
        plsc.subcore_barrier()
        pltpu.sync_copy(acc.at[pl.ds(r0, rps)],
                        out_hbm.at[pl.ds(cid * npad + r0, rps)])

    return k(tab2, de, src, dst, zeros)


def _pool_call(x, cl, zeros, npad_in, npad_out):
    perw = npad_in // NW
    iters = perw // HC
    rps = npad_out // NS

    @functools.partial(
        pl.kernel, mesh=_mesh(),
        out_type=jax.ShapeDtypeStruct((2 * npad_out, C), F32),
        scratch_types=[
            pltpu.VMEM((HC,), I32),
            pltpu.VMEM((HC, C), F32),
            pltpu.VMEM_SHARED((npad_out, C), F32),
            pltpu.SemaphoreType.DMA,
        ],
        compiler_params=_SC_PARAMS,
    )
    def k(x_hbm, cl_hbm, zer_hbm, out_hbm, clv, rowsv, acc, sem):
        cid = lax.axis_index("c")
        sid = lax.axis_index("s")
        wid = sid * NC + cid
        r0 = sid * rps
        pltpu.sync_copy(zer_hbm, acc.at[pl.ds(r0, rps)])
        plsc.subcore_barrier()
        base = wid * perw

        def body(i, carry):
            off = base + i * HC
            pltpu.sync_copy(cl_hbm.at[pl.ds(off, HC)], clv)
            pltpu.sync_copy(x_hbm.at[pl.ds(off, HC)], rowsv)
            pltpu.sync_copy(rowsv, acc.at[clv], add=True)
            return carry

        lax.fori_loop(0, iters, body, 0)
        plsc.subcore_barrier()
        pltpu.sync_copy(acc.at[pl.ds(r0, rps)],
                        out_hbm.at[pl.ds(cid * npad_out + r0, rps)])

    return k(x, cl, zeros)


def _unpool_call(table, idx, npad_out, npad_in):
    perw = npad_out // NW
    iters = perw // HC

    @functools.partial(
        pl.kernel, mesh=_mesh(),
        out_type=jax.ShapeDtypeStruct((npad_out, C), F32),
        scratch_types=[
            pltpu.VMEM((HC,), I32),
            pltpu.VMEM((HC, C), F32),
            pltpu.SemaphoreType.DMA,
        ],
        compiler_params=_SC_PARAMS,
    )
    def k(tab_hbm, idx_hbm, out_hbm, idxv, rowsv, sem):
        cid = lax.axis_index("c")
        sid = lax.axis_index("s")
        wid = sid * NC + cid
        base = wid * perw

        def body(i, carry):
            off = base + i * HC
            pltpu.sync_copy(idx_hbm.at[pl.ds(off, HC)], idxv)
            pltpu.async_copy(tab_hbm.at[idxv], rowsv, sem).wait()
            pltpu.sync_copy(rowsv, out_hbm.at[pl.ds(off, HC)])
            return carry

        lax.fori_loop(0, iters, body, 0)

    return k(table, idx)


def _full(shape):
    return pl.BlockSpec(shape, lambda i: (0,) * len(shape))


def _rows(w):
    return pl.BlockSpec((BLK, w), lambda i: (i, 0))


def _tc_direct_call(xa, xb, walla, wallb, wua, wub, w2a, w2b, npad):
    dual = xb is not None

    def body(*refs):
        if dual:
            (xa_r, xb_r, wa_r, wb_r, wua_r, wub_r, w2a_r, w2b_r,
             t_r, l_r, s_r) = refs
        else:
            xa_r, wa_r, wua_r, w2a_r, t_r, l_r, s_r = refs
        x = xa_r[...]
        y = jnp.dot(x, wa_r[...], preferred_element_type=F32)
        l8 = jnp.dot(x, wua_r[...], preferred_element_type=F32)
        sk = jnp.dot(x, w2a_r[...], preferred_element_type=F32)
        if dual:
            x2 = xb_r[...]
            y = y + jnp.dot(x2, wb_r[...], preferred_element_type=F32)
            l8 = l8 + jnp.dot(x2, wub_r[...], preferred_element_type=F32)
            sk = sk + jnp.dot(x2, w2b_r[...], preferred_element_type=F32)
        t_r[0] = y[:, 0:TW]
        t_r[1] = y[:, TW:2 * TW]
        l_r[...] = l8
        s_r[...] = sk

    ins = [xa] + ([xb] if dual else []) + [walla] + ([wallb] if dual else []) \
        + [wua] + ([wub] if dual else []) + [w2a] + ([w2b] if dual else [])
    in_specs = [_rows(C)] + ([_rows(C)] if dual else []) \
        + [_full((C, 2 * TW))] + ([_full((C, 2 * TW))] if dual else []) \
        + [_full((C, 8))] + ([_full((C, 8))] if dual else []) \
        + [_full((C, C))] + ([_full((C, C))] if dual else [])
    return pl.pallas_call(
        body,
        grid=(npad // BLK,),
        in_specs=in_specs,
        out_specs=[pl.BlockSpec((2, BLK, TW), lambda i: (0, i, 0)),
                   _rows(8), _rows(C)],
        out_shape=[
            jax.ShapeDtypeStruct((2, npad, TW), F32),
            jax.ShapeDtypeStruct((npad, 8), F32),
            jax.ShapeDtypeStruct((npad, C), F32),
        ],
    )(*ins)


def _tc_combine_call(pflat, scale8, bias8, skip, wall, wu, w2, relu, matmul,
                     has_w2, pool, npad):
    has_skip = skip is not None
    nb = npad // BLK

    def body(*refs):
        i = 0
        p0_r = refs[i]; i += 1
        p1_r = refs[i]; i += 1
        sc_r = refs[i]; i += 1
        b_r = refs[i]; i += 1
        sk_r = None
        if has_skip:
            sk_r = refs[i]; i += 1
        if pool:
            agg = p0_r[...] + p1_r[...]
        else:
            agg = jnp.concatenate([p0_r[...][:, 0:CH], p1_r[...][:, 0:CH]],
                                  axis=1)
        x = agg * sc_r[...][:, 0:1] + b_r[0:1, :]
        if has_skip:
            x = x + sk_r[...]
        if relu:
            x = jnp.maximum(x, 0.0)
        if matmul:
            w_r = refs[i]; i += 1
            wu_r = refs[i]; i += 1
            w2_r = None
            if has_w2:
                w2_r = refs[i]; i += 1
            xo_r = refs[i]; i += 1
            t_r = refs[i]; i += 1
            l_r = refs[i]; i += 1
            xo_r[...] = x
            y = jnp.dot(x, w_r[...], preferred_element_type=F32)
            t_r[0] = y[:, 0:TW]
            t_r[1] = y[:, TW:2 * TW]
            l_r[...] = jnp.dot(x, wu_r[...], preferred_element_type=F32)
            if has_w2:
                s_r = refs[i]
                s_r[...] = jnp.dot(x, w2_r[...], preferred_element_type=F32)
        else:
            xo_r = refs[i]
            xo_r[...] = x

    ins = [pflat, pflat, scale8, bias8]
    in_specs = [
        pl.BlockSpec((BLK, C), lambda i: (i, 0)),
        pl.BlockSpec((BLK, C), lambda i: (i + nb, 0)),
        pl.BlockSpec((BLK, 8), lambda i: (i, 0)),
        _full((8, C)),
    ]
    if has_skip:
        ins.append(skip)
        in_specs.append(_rows(C))
    out_specs = [_rows(C)]
    out_shape = [jax.ShapeDtypeStruct((npad, C), F32)]
    if matmul:
        ins += [wall, wu] + ([w2] if has_w2 else [])
        in_specs += [_full((C, 2 * TW)), _full((C, 8))] \
            + ([_full((C, C))] if has_w2 else [])
        out_specs += [pl.BlockSpec((2, BLK, TW), lambda i: (0, i, 0)),
                      _rows(8)] + ([_rows(C)] if has_w2 else [])
        out_shape += [jax.ShapeDtypeStruct((2, npad, TW), F32),
                      jax.ShapeDtypeStruct((npad, 8), F32)] \
            + ([jax.ShapeDtypeStruct((npad, C), F32)] if has_w2 else [])
    return pl.pallas_call(
        body,
        grid=(nb,),
        in_specs=in_specs,
        out_specs=out_specs,
        out_shape=out_shape,
    )(*ins)


def _pad2(a, r, c):
    return jnp.pad(a, ((0, r - a.shape[0]), (0, c - a.shape[1])))


def _pack_wall(p, lo, hi):
    w = p["W"][lo:hi]
    u = p["u"][lo:hi]
    cout = p["b"].shape[0]
    y0 = _pad2(w[:, :cout], C, C)
    y1 = _pad2(w[:, cout:], C, C)
    ud = jnp.pad(u[:, 0] - u[:, 1], (0, C - u.shape[0]))
    wu = jnp.zeros((C, 8), F32).at[:, 0].set(ud)
    blkA = jnp.concatenate([y0[:, 0:CH], y1[:, 0:CH]], axis=1)
    blkB = jnp.concatenate([y0[:, CH:C], y1[:, CH:C]], axis=1)
    wall = jnp.concatenate([blkA, blkB], axis=1)
    w2 = _pad2(p["Ws"][lo:hi], C, C) if "Ws" in p else None
    return wall, wu, w2


def _bias8(p):
    return jnp.broadcast_to(jnp.pad(p["b"], (0, C - p["b"].shape[0])), (8, C))


def _pad_idx(idx, tot, lo, hi):
    n = idx.shape[0]
    fill = lo + (jnp.arange(tot - n, dtype=I32) % (hi - lo))
    return jnp.concatenate([idx.astype(I32), fill])


def kernel(feat, geo, params, scale0_edge_index, edge_index1, edge_index2,
           cluster1, cluster2):
    rows = jnp.array([0, 0, 0, 1, 1, 2])
    cols = jnp.array([0, 1, 2, 1, 2, 2])
    t0 = feat[:, 0][:, rows, cols]
    t1 = feat[:, 1][:, rows, cols]
    t2 = feat[:, 2].reshape(-1, 9)
    x0 = jnp.concatenate([t0, t1, t2, geo[:, None]], axis=1)
    x0 = jnp.pad(x0, ((0, NP0 - NB0), (0, C - x0.shape[1])))

    egrp = 2 * NS * EC
    ep0 = egrp * -(-scale0_edge_index.shape[1] // egrp)
    ep1 = egrp * -(-edge_index1.shape[1] // egrp)
    ep2 = egrp * -(-edge_index2.shape[1] // egrp)
    src0 = _pad_idx(scale0_edge_index[0], ep0, 0, NB0)
    dst0 = _pad_idx(scale0_edge_index[1], ep0, NB0, NP0)
    src1 = _pad_idx(edge_index1[0], ep1, 0, NB1)
    dst1 = _pad_idx(edge_index1[1], ep1, NB1, NP1)
    src2 = _pad_idx(edge_index2[0], ep2, 0, NB2)
    dst2 = _pad_idx(edge_index2[1], ep2, NB2, NP2)
    c1p = _pad_idx(cluster1, NP0, NB1, NP1)
    c2p = _pad_idx(cluster2, NP1, NB2, NP2)
    u1p = _pad_idx(cluster2, NP1, 0, NB2)
    u0p = _pad_idx(cluster1, NP0, 0, NB1)

    ones_h = jnp.ones((HC, C), F32)
    z8 = {n: jnp.zeros((n // NS, C), F32) for n in (NP0, NP1, NP2)}
    zF = {n: jnp.zeros((n // NS, C), F32) for n in (NP0, NP1, NP2)}

    hp0 = NW * HC * -(-ep0 // (NW * HC))
    hp1 = NW * HC * -(-ep1 // (NW * HC))
    hp2 = NW * HC * -(-ep2 // (NW * HC))
    dst0h = _pad_idx(dst0, hp0, NB0, NP0)
    dst1h = _pad_idx(dst1, hp1, NB1, NP1)
    dst2h = _pad_idx(dst2, hp2, NB2, NP2)
    invd0 = _recip_call(_hist_call(dst0h, ones_h, z8[NP0], hp0, NP0), NP0)
    invd1 = _recip_call(_hist_call(dst1h, ones_h, z8[NP1], hp1, NP1), NP1)
    invd2 = _recip_call(_hist_call(dst2h, ones_h, z8[NP2], hp2, NP2), NP2)
    invc1 = _recip_call(_hist_call(c1p, ones_h, z8[NP1], NP0, NP1), NP1)
    invc2 = _recip_call(_hist_call(c2p, ones_h, z8[NP2], NP1, NP2), NP2)

    scale = {0: (src0, dst0, ep0, NP0, invd0),
             1: (src1, dst1, ep1, NP1, invd1),
             2: (src2, dst2, ep2, NP2, invd2)}

    def edge(name, tab, l8, s):
        src, dst, ep, npad, _ = scale[s]
        p = params[name]
        cdiff = p["c"][0] - p["c"][1]
        l = l8[:, 0]
        de = l[src] - l[dst] + cdiff
        return _edge_call(tab.reshape(2 * npad, TW), de, src, dst,
                          zF[npad], npad, ep)

    def comb(pf, name, s, skip, relu, matmul, has_w2=False, wname=None):
        _, _, _, npad, invd = scale[s]
        b8 = _bias8(params[name])
        if matmul:
            wall, wu, w2 = _pack_wall(params[wname], 0, C)
            if not has_w2:
                w2 = None
            return _tc_combine_call(pf, invd, b8, skip, wall, wu, w2, relu,
                                    True, has_w2, False, npad)
        return _tc_combine_call(pf, invd, b8, skip, None, None, None, relu,
                                False, False, False, npad)

    def pool_comb(pf, invc, wname, npad):
        wall, wu, _ = _pack_wall(params[wname], 0, C)
        return _tc_combine_call(pf, invc, zb, None, wall, wu, None, False,
                                True, False, True, npad)

    zb = jnp.zeros((8, C), F32)

    wall, wu, w2 = _pack_wall(params["conv01"], 0, C)
    tab, l8, sky = _tc_direct_call(x0, None, wall, None, wu, None, w2, None,
                                   NP0)
    pf = edge("conv01", tab, l8, 0)
    x1, tab, l8 = comb(pf, "conv01", 0, sky, True, True, wname="conv02")
    pf = edge("conv02", tab, l8, 0)
    copy0, = comb(pf, "conv02", 0, x1, True, False)

    pfp = _pool_call(copy0, c1p, zF[NP1], NP0, NP1)
    xp1, tab, l8 = pool_comb(pfp, invc1, "conv11", NP1)
    pf = edge("conv11", tab, l8, 1)
    x11, tab, l8 = comb(pf, "conv11", 1, xp1, True, True, wname="conv12")
    pf = edge("conv12", tab, l8, 1)
    copy1, = comb(pf, "conv12", 1, x11, True, False)

    pfp = _pool_call(copy1, c2p, zF[NP2], NP1, NP2)
    xp2, tab, l8 = pool_comb(pfp, invc2, "conv21", NP2)
    pf = edge("conv21", tab, l8, 2)
    x21, tab, l8 = comb(pf, "conv21", 2, xp2, True, True, wname="conv22")
    pf = edge("conv22", tab, l8, 2)
    x2f, = comb(pf, "conv22", 2, x21, True, False)

    xu1 = _unpool_call(x2f, u1p, NP1, NP2)
    wa, wua, w2a = _pack_wall(params["conv13"], 0, 115)
    wb, wub, w2b = _pack_wall(params["conv13"], 115, 230)
    tab, l8, sky = _tc_direct_call(xu1, copy1, wa, wb, wua, wub, w2a, w2b,
                                   NP1)
    pf = edge("conv13", tab, l8, 1)
    x13, tab, l8 = comb(pf, "conv13", 1, sky, True, True, wname="conv14")
    pf = edge("conv14", tab, l8, 1)
    x14, tab, l8 = comb(pf, "conv14", 1, x13, True, True, wname="conv15")
    pf = edge("conv15", tab, l8, 1)
    x15, tab, l8 = comb(pf, "conv15", 1, x14, True, True, wname="conv16")
    pf = edge("conv16", tab, l8, 1)
    x1f, = comb(pf, "conv16", 1, x15, True, False)

    xu0 = _unpool_call(x1f, u0p, NP0, NP1)
    wa, wua, w2a = _pack_wall(params["conv03"], 0, 115)
    wb, wub, w2b = _pack_wall(params["conv03"], 115, 230)
    tab, l8, sky = _tc_direct_call(xu0, copy0, wa, wb, wua, wub, w2a, w2b,
                                   NP0)
    pf = edge("conv03", tab, l8, 0)
    x03, tab, l8 = comb(pf, "conv03", 0, sky, True, True, wname="conv04")
    pf = edge("conv04", tab, l8, 0)
    x04, tab, l8 = comb(pf, "conv04", 0, x03, True, True, wname="conv05")
    pf = edge("conv05", tab, l8, 0)
    x05, tab, l8, sky = comb(pf, "conv05", 0, x04, True, True, has_w2=True,
                             wname="conv06")
    pf = edge("conv06", tab, l8, 0)
    out, = comb(pf, "conv06", 0, sky, False, False)
    return out[:NB0, :3]
